# baseline (speedup 1.0000x reference)
import functools

import jax
import jax.numpy as jnp
from jax import lax
from jax.experimental import pallas as pl
from jax.experimental.pallas import tpu as pltpu
from jax.experimental.pallas import tpu_sc as plsc

N = 10000
E = 320000
D_FEAT = 128
NUM_GRAPHS = 64
NUM_CLASSES = 10
H0 = 64
H1 = 32
H_MLP = 128

NCORE = 2
NSUB = 16
NW = NCORE * NSUB
NP = 10240
EP = 327680
EW_CNT = EP // NW
CH = 128
NCH = EW_CNT // CH
ZR = 32

_mesh = plsc.VectorSubcoreMesh(core_axis_name="c", subcore_axis_name="s")


@functools.partial(
    pl.kernel,
    out_type=jax.ShapeDtypeStruct((NCORE, NP), jnp.float32),
    mesh=_mesh,
    scratch_types=[
        pltpu.VMEM((NCH, CH), jnp.int32),
        pltpu.VMEM((NCH, CH), jnp.float32),
        pltpu.VMEM((NP // NSUB,), jnp.float32),
        pltpu.VMEM_SHARED((NP,), jnp.float32),
        pltpu.SemaphoreType.DMA,
    ],
)
def _sc_degree(row_hbm, ew_hbm, out_hbm, rowm, ewm, zbuf, acc, sem):
    c = lax.axis_index("c")
    s = lax.axis_index("s")
    wid = c * NSUB + s
    slab = NP // NSUB

    pltpu.sync_copy(row_hbm.at[wid], rowm)
    pltpu.sync_copy(ew_hbm.at[wid], ewm)

    def zb(i, _):
        zbuf[pl.ds(i * 16, 16)] = jnp.zeros((16,), jnp.float32)
        return 0

    lax.fori_loop(0, slab // 16, zb, 0)
    pltpu.sync_copy(zbuf, acc.at[pl.ds(s * slab, slab)])
    plsc.subcore_barrier()

    W = 8

    def _issue(g):
        pltpu.async_copy(ewm.at[g], acc.at[rowm.at[g]], sem, add=True)

    def _wait(g):
        pltpu.make_async_copy(ewm.at[g], acc.at[rowm.at[g]], sem).wait()

    for g in range(W):
        _issue(g)

    def deg_body(g, _):
        _issue(g)
        _wait(g - W)
        return 0

    lax.fori_loop(W, NCH, deg_body, 0)
    for g in range(NCH - W, NCH):
        _wait(g)
    plsc.subcore_barrier()
    pltpu.sync_copy(acc.at[pl.ds(s * slab, slab)], out_hbm.at[c, pl.ds(s * slab, slab)])


NBUF = 5
GA = 3
SBUF = 2
RND = 10


def _make_sc_edge(H):
    H2 = H // 2

    @functools.partial(
        pl.kernel,
        out_type=jax.ShapeDtypeStruct((NCORE, NP, H), jnp.float32),
        mesh=_mesh,
        scratch_types=[
            pltpu.VMEM((NCH, CH), jnp.int32),
            pltpu.VMEM((NCH, CH), jnp.int32),
            pltpu.VMEM((NCH, CH), jnp.float32),
            pltpu.VMEM((NBUF, CH, H2), jnp.uint32),
            pltpu.VMEM((SBUF, CH, H), jnp.float32),
            pltpu.VMEM((ZR, H), jnp.float32),
            pltpu.VMEM_SHARED((NP, H), jnp.float32),
        ] + [pltpu.SemaphoreType.DMA] * (NBUF + SBUF),
        compiler_params=pltpu.CompilerParams(use_tc_tiling_on_sc=False,
                                             needs_layout_passes=False),
    )
    def _sc_edge(h_hbm, row_hbm, col_hbm, ew_hbm, out_hbm, rowm, colm, ewm,
                 msg, msgf, zbuf, acc, *sems):
        gsem = sems[:NBUF]
        ssem = sems[NBUF:]
        c = lax.axis_index("c")
        s = lax.axis_index("s")
        wid = c * NSUB + s
        slab = NP // NSUB

        pltpu.sync_copy(row_hbm.at[wid], rowm)
        pltpu.sync_copy(col_hbm.at[wid], colm)
        pltpu.sync_copy(ew_hbm.at[wid], ewm)

        def zb(j, _):
            for k in range(H // 16):
                zbuf[j, pl.ds(k * 16, 16)] = jnp.zeros((16,), jnp.float32)
            return 0

        lax.fori_loop(0, ZR, zb, 0)

        def zfill(j, _):
            pltpu.sync_copy(zbuf, acc.at[pl.ds(s * slab + j * ZR, ZR), :])
            return 0

        lax.fori_loop(0, slab // ZR, zfill, 0)
        plsc.subcore_barrier()

        def issue_gather(g, b):
            pltpu.async_copy(h_hbm.at[colm.at[g]], msg.at[b], gsem[b])

        def wait_gather(g, b):
            pltpu.make_async_copy(h_hbm.at[colm.at[g]], msg.at[b],
                                  gsem[b]).wait()

        def issue_scatter(g, sb):
            pltpu.async_copy(msgf.at[sb], acc.at[rowm.at[g]], ssem[sb],
                             add=True)

        def wait_scatter(g, sb):
            pltpu.make_async_copy(msgf.at[sb], acc.at[rowm.at[g]],
                                  ssem[sb]).wait()

        hi_mask = jnp.uint32(0xFFFF0000)

        def scale(g, b, sb):
            def body(j, _):
                w16 = ewm[g, pl.ds(j * 16, 16)]
                for t in range(16):
                    w = w16[t]
                    e = j * 16 + t
                    for k in range(H2 // 16):
                        cvec = msg[b, e, pl.ds(k * 16, 16)]
                        lo = plsc.bitcast(cvec << 16, jnp.float32)
                        hi = plsc.bitcast(cvec & hi_mask, jnp.float32)
                        msgf[sb, e, pl.ds(k * 32, 16)] = lo * w
                        msgf[sb, e, pl.ds(k * 32 + 16, 16)] = hi * w
                return 0

            lax.fori_loop(0, CH // 16, body, 0)

        def step(g, b, sb, do_wait_scatter, do_issue_gather, bf):
            if do_wait_scatter:
                wait_scatter(g - SBUF, sb)
            wait_gather(g, b)
            scale(g, b, sb)
            issue_scatter(g, sb)
            if do_issue_gather:
                issue_gather(g + GA, bf)

        for g in range(GA):
            issue_gather(g, g)
        for g in range(RND):
            step(g, g % NBUF, g % SBUF, g >= SBUF, True, (g + GA) % NBUF)

        def round_body(i, _):
            g0 = i * RND
            for k in range(RND):
                step(g0 + k, k % NBUF, k % SBUF, True, True,
                     (k + GA) % NBUF)
            return 0

        lax.fori_loop(1, NCH // RND - 1, round_body, 0)

        for g in range(NCH - RND, NCH):
            step(g, g % NBUF, g % SBUF, True, g + GA < NCH, (g + GA) % NBUF)
        for g in range(NCH - SBUF, NCH):
            wait_scatter(g, g % SBUF)
        plsc.subcore_barrier()
        pltpu.sync_copy(acc.at[pl.ds(s * slab, slab), :],
                        out_hbm.at[c, pl.ds(s * slab, slab), :])

    return _sc_edge


_sc_edge64 = _make_sc_edge(H0)
_sc_edge32 = _make_sc_edge(H1)


def _tc_stage1_body(deg0_ref, deg1_ref, x_ref, w0_ref, h0s_ref, dinv_ref):
    deg = deg0_ref[...] + deg1_ref[...] + 1.0
    dinv = lax.rsqrt(jnp.maximum(deg, 1e-12))
    dinv_ref[...] = dinv
    h0 = jnp.dot(x_ref[...], w0_ref[...], preferred_element_type=jnp.float32)
    h0s_ref[...] = (h0 * dinv).astype(jnp.bfloat16)


def _tc_stage1(deg0, deg1, x, W0):
    return pl.pallas_call(
        _tc_stage1_body,
        out_shape=(
            jax.ShapeDtypeStruct((N, H0), jnp.bfloat16),
            jax.ShapeDtypeStruct((N, 1), jnp.float32),
        ),
    )(deg0, deg1, x, W0)


def _tc_stage2_body(p0_ref, p1_ref, h0s_ref, dinv_ref, b0_ref, w1_ref, h1s_ref):
    dinv = dinv_ref[...]
    h0s = h0s_ref[...].astype(jnp.float32)
    a1 = jnp.maximum(
        dinv * (p0_ref[...] + p1_ref[...] + h0s) + b0_ref[...], 0.0)
    h1 = jnp.dot(a1, w1_ref[...], preferred_element_type=jnp.float32)
    h1s_ref[...] = (h1 * dinv).astype(jnp.bfloat16)


def _tc_stage2(p0, p1, h0s, dinv, b0, W1):
    return pl.pallas_call(
        _tc_stage2_body,
        out_shape=jax.ShapeDtypeStruct((N, H1), jnp.bfloat16),
    )(p0, p1, h0s, dinv, b0, W1)


def _tc_stage3_body(q0_ref, q1_ref, h1s_ref, dinv_ref, b1_ref, ngi_ref,
                    wm1_ref, bm1_ref, wm2_ref, bm2_ref, out_ref):
    dinv = dinv_ref[...]
    h1s = h1s_ref[...].astype(jnp.float32)
    a2 = jnp.maximum(
        dinv * (q0_ref[...] + q1_ref[...] + h1s) + b1_ref[...], 0.0)
    gids = lax.broadcasted_iota(jnp.int32, (N, NUM_GRAPHS), 1)
    onehot = (ngi_ref[...] == gids).astype(jnp.float32)
    pooled = lax.dot_general(onehot, a2, (((0,), (0,)), ((), ())),
                             preferred_element_type=jnp.float32)
    h2 = jnp.maximum(
        jnp.dot(pooled, wm1_ref[...], preferred_element_type=jnp.float32)
        + bm1_ref[...], 0.0)
    out_ref[...] = jnp.dot(h2, wm2_ref[...],
                           preferred_element_type=jnp.float32) + bm2_ref[...]


def _tc_stage3(q0, q1, h1s, dinv, b1, ngi, Wm1, bm1, Wm2, bm2):
    return pl.pallas_call(
        _tc_stage3_body,
        out_shape=jax.ShapeDtypeStruct((NUM_GRAPHS, NUM_CLASSES), jnp.float32),
    )(q0, q1, h1s, dinv, b1, ngi, Wm1, bm1, Wm2, bm2)


def _pack_pairs(h):
    n, h_dim = h.shape
    hp = h.reshape(n, h_dim // 32, 2, 16).transpose(0, 1, 3, 2)
    return jax.lax.bitcast_convert_type(hp.reshape(n, h_dim // 2, 2),
                                        jnp.uint32)


@jax.jit
def kernel(x, edge_index, edge_weight, node_graph_index,
           W0, b0, W1, b1, Wm1, bm1, Wm2, bm2):
    pad = EP - E
    row = jnp.concatenate([edge_index[0], jnp.zeros((pad,), jnp.int32)])
    col = jnp.concatenate([edge_index[1], jnp.zeros((pad,), jnp.int32)])
    ew = jnp.concatenate([edge_weight, jnp.zeros((pad,), jnp.float32)])
    row = row.reshape(NW, NCH, CH)
    col = col.reshape(NW, NCH, CH)
    ew = ew.reshape(NW, NCH, CH)

    degp = _sc_degree(row, ew)
    deg0 = degp[0, :N].reshape(N, 1)
    deg1 = degp[1, :N].reshape(N, 1)

    h0s, dinv = _tc_stage1(deg0, deg1, x, W0)

    p = _sc_edge64(_pack_pairs(h0s), row, col, ew)
    h1s = _tc_stage2(p[0, :N], p[1, :N], h0s, dinv, b0.reshape(1, H0), W1)

    q = _sc_edge32(_pack_pairs(h1s), row, col, ew)
    logits = _tc_stage3(q[0, :N], q[1, :N], h1s, dinv, b1.reshape(1, H1),
                        node_graph_index.reshape(N, 1),
                        Wm1, bm1.reshape(1, H_MLP), Wm2, bm2.reshape(1, NUM_CLASSES))
    return logits


# speedup vs baseline: 1.0133x; 1.0133x over previous
import functools

import jax
import jax.numpy as jnp
from jax import lax
from jax.experimental import pallas as pl
from jax.experimental.pallas import tpu as pltpu
from jax.experimental.pallas import tpu_sc as plsc

N = 10000
E = 320000
D_FEAT = 128
NUM_GRAPHS = 64
NUM_CLASSES = 10
H0 = 64
H1 = 32
H_MLP = 128

NCORE = 2
NSUB = 16
NW = NCORE * NSUB
NP = 10240
EP = 327680
EW_CNT = EP // NW
CH = 128
NCH = EW_CNT // CH
ZR = 128

_mesh = plsc.VectorSubcoreMesh(core_axis_name="c", subcore_axis_name="s")


@functools.partial(
    pl.kernel,
    out_type=jax.ShapeDtypeStruct((NCORE, NP), jnp.float32),
    mesh=_mesh,
    scratch_types=[
        pltpu.VMEM((NCH, CH), jnp.int32),
        pltpu.VMEM((NCH, CH), jnp.float32),
        pltpu.VMEM((NP // NSUB,), jnp.float32),
        pltpu.VMEM_SHARED((NP,), jnp.float32),
        pltpu.SemaphoreType.DMA,
    ],
)
def _sc_degree(row_hbm, ew_hbm, out_hbm, rowm, ewm, zbuf, acc, sem):
    c = lax.axis_index("c")
    s = lax.axis_index("s")
    wid = c * NSUB + s
    slab = NP // NSUB

    pltpu.sync_copy(row_hbm.at[wid], rowm)
    pltpu.sync_copy(ew_hbm.at[wid], ewm)

    def zb(i, _):
        zbuf[pl.ds(i * 16, 16)] = jnp.zeros((16,), jnp.float32)
        return 0

    lax.fori_loop(0, slab // 16, zb, 0)
    pltpu.sync_copy(zbuf, acc.at[pl.ds(s * slab, slab)])
    plsc.subcore_barrier()

    W = 8

    def _issue(g):
        pltpu.async_copy(ewm.at[g], acc.at[rowm.at[g]], sem, add=True)

    def _wait(g):
        pltpu.make_async_copy(ewm.at[g], acc.at[rowm.at[g]], sem).wait()

    for g in range(W):
        _issue(g)

    def deg_body(g, _):
        _issue(g)
        _wait(g - W)
        return 0

    lax.fori_loop(W, NCH, deg_body, 0)
    for g in range(NCH - W, NCH):
        _wait(g)
    plsc.subcore_barrier()
    pltpu.sync_copy(acc.at[pl.ds(s * slab, slab)], out_hbm.at[c, pl.ds(s * slab, slab)])


NBUF = 5
GA = 4
SBUF = 2
RND = 10


def _make_sc_edge(H):
    H2 = H // 2

    @functools.partial(
        pl.kernel,
        out_type=jax.ShapeDtypeStruct((NCORE, NP, H), jnp.float32),
        mesh=_mesh,
        scratch_types=[
            pltpu.VMEM((NCH, CH), jnp.int32),
            pltpu.VMEM((NCH, CH), jnp.int32),
            pltpu.VMEM((NCH, CH), jnp.float32),
            pltpu.VMEM((NBUF, CH, H2), jnp.uint32),
            pltpu.VMEM((SBUF, CH, H), jnp.float32),
            pltpu.VMEM((ZR, H), jnp.float32),
            pltpu.VMEM_SHARED((NP, H), jnp.float32),
        ] + [pltpu.SemaphoreType.DMA] * (NBUF + SBUF),
        compiler_params=pltpu.CompilerParams(use_tc_tiling_on_sc=False,
                                             needs_layout_passes=False),
    )
    def _sc_edge(h_hbm, row_hbm, col_hbm, ew_hbm, out_hbm, rowm, colm, ewm,
                 msg, msgf, zbuf, acc, *sems):
        gsem = sems[:NBUF]
        ssem = sems[NBUF:]
        c = lax.axis_index("c")
        s = lax.axis_index("s")
        wid = c * NSUB + s
        slab = NP // NSUB

        pltpu.sync_copy(row_hbm.at[wid], rowm)
        pltpu.sync_copy(col_hbm.at[wid], colm)
        pltpu.sync_copy(ew_hbm.at[wid], ewm)

        def zb(j, _):
            for k in range(H // 16):
                zbuf[j, pl.ds(k * 16, 16)] = jnp.zeros((16,), jnp.float32)
            return 0

        lax.fori_loop(0, ZR, zb, 0)

        for j in range(slab // ZR):
            pltpu.async_copy(zbuf, acc.at[pl.ds(s * slab + j * ZR, ZR), :],
                             ssem[0])
        for j in range(slab // ZR):
            pltpu.make_async_copy(
                zbuf, acc.at[pl.ds(s * slab + j * ZR, ZR), :], ssem[0]).wait()
        plsc.subcore_barrier()

        def issue_gather(g, b):
            pltpu.async_copy(h_hbm.at[colm.at[g]], msg.at[b], gsem[b])

        def wait_gather(g, b):
            pltpu.make_async_copy(h_hbm.at[colm.at[g]], msg.at[b],
                                  gsem[b]).wait()

        def issue_scatter(g, sb):
            pltpu.async_copy(msgf.at[sb], acc.at[rowm.at[g]], ssem[sb],
                             add=True)

        def wait_scatter(g, sb):
            pltpu.make_async_copy(msgf.at[sb], acc.at[rowm.at[g]],
                                  ssem[sb]).wait()

        hi_mask = jnp.uint32(0xFFFF0000)

        def scale(g, b, sb):
            def body(j, _):
                w16 = ewm[g, pl.ds(j * 16, 16)]
                for t in range(16):
                    w = w16[t]
                    e = j * 16 + t
                    for k in range(H2 // 16):
                        cvec = msg[b, e, pl.ds(k * 16, 16)]
                        lo = plsc.bitcast(cvec << 16, jnp.float32)
                        hi = plsc.bitcast(cvec & hi_mask, jnp.float32)
                        msgf[sb, e, pl.ds(k * 32, 16)] = lo * w
                        msgf[sb, e, pl.ds(k * 32 + 16, 16)] = hi * w
                return 0

            lax.fori_loop(0, CH // 16, body, 0)

        def step(g, b, sb, do_wait_scatter, do_issue_gather, bf):
            if do_wait_scatter:
                wait_scatter(g - SBUF, sb)
            wait_gather(g, b)
            scale(g, b, sb)
            issue_scatter(g, sb)
            if do_issue_gather:
                issue_gather(g + GA, bf)

        for g in range(GA):
            issue_gather(g, g)
        for g in range(RND):
            step(g, g % NBUF, g % SBUF, g >= SBUF, True, (g + GA) % NBUF)

        def round_body(i, _):
            g0 = i * RND
            for k in range(RND):
                step(g0 + k, k % NBUF, k % SBUF, True, True,
                     (k + GA) % NBUF)
            return 0

        lax.fori_loop(1, NCH // RND - 1, round_body, 0)

        for g in range(NCH - RND, NCH):
            step(g, g % NBUF, g % SBUF, True, g + GA < NCH, (g + GA) % NBUF)
        for g in range(NCH - SBUF, NCH):
            wait_scatter(g, g % SBUF)
        plsc.subcore_barrier()
        pltpu.sync_copy(acc.at[pl.ds(s * slab, slab), :],
                        out_hbm.at[c, pl.ds(s * slab, slab), :])

    return _sc_edge


_sc_edge64 = _make_sc_edge(H0)
_sc_edge32 = _make_sc_edge(H1)


def _tc_stage1_body(deg0_ref, deg1_ref, x_ref, w0_ref, h0s_ref, dinv_ref):
    deg = deg0_ref[...] + deg1_ref[...] + 1.0
    dinv = lax.rsqrt(jnp.maximum(deg, 1e-12))
    dinv_ref[...] = dinv
    h0 = jnp.dot(x_ref[...], w0_ref[...], preferred_element_type=jnp.float32)
    h0s_ref[...] = (h0 * dinv).astype(jnp.bfloat16)


def _tc_stage1(deg0, deg1, x, W0):
    return pl.pallas_call(
        _tc_stage1_body,
        out_shape=(
            jax.ShapeDtypeStruct((N, H0), jnp.bfloat16),
            jax.ShapeDtypeStruct((N, 1), jnp.float32),
        ),
    )(deg0, deg1, x, W0)


def _tc_stage2_body(p0_ref, p1_ref, h0s_ref, dinv_ref, b0_ref, w1_ref, h1s_ref):
    dinv = dinv_ref[...]
    h0s = h0s_ref[...].astype(jnp.float32)
    a1 = jnp.maximum(
        dinv * (p0_ref[...] + p1_ref[...] + h0s) + b0_ref[...], 0.0)
    h1 = jnp.dot(a1, w1_ref[...], preferred_element_type=jnp.float32)
    h1s_ref[...] = (h1 * dinv).astype(jnp.bfloat16)


def _tc_stage2(p0, p1, h0s, dinv, b0, W1):
    return pl.pallas_call(
        _tc_stage2_body,
        out_shape=jax.ShapeDtypeStruct((N, H1), jnp.bfloat16),
    )(p0, p1, h0s, dinv, b0, W1)


def _tc_stage3_body(q0_ref, q1_ref, h1s_ref, dinv_ref, b1_ref, ngi_ref,
                    wm1_ref, bm1_ref, wm2_ref, bm2_ref, out_ref):
    dinv = dinv_ref[...]
    h1s = h1s_ref[...].astype(jnp.float32)
    a2 = jnp.maximum(
        dinv * (q0_ref[...] + q1_ref[...] + h1s) + b1_ref[...], 0.0)
    gids = lax.broadcasted_iota(jnp.int32, (N, NUM_GRAPHS), 1)
    onehot = (ngi_ref[...] == gids).astype(jnp.float32)
    pooled = lax.dot_general(onehot, a2, (((0,), (0,)), ((), ())),
                             preferred_element_type=jnp.float32)
    h2 = jnp.maximum(
        jnp.dot(pooled, wm1_ref[...], preferred_element_type=jnp.float32)
        + bm1_ref[...], 0.0)
    out_ref[...] = jnp.dot(h2, wm2_ref[...],
                           preferred_element_type=jnp.float32) + bm2_ref[...]


def _tc_stage3(q0, q1, h1s, dinv, b1, ngi, Wm1, bm1, Wm2, bm2):
    return pl.pallas_call(
        _tc_stage3_body,
        out_shape=jax.ShapeDtypeStruct((NUM_GRAPHS, NUM_CLASSES), jnp.float32),
    )(q0, q1, h1s, dinv, b1, ngi, Wm1, bm1, Wm2, bm2)


def _pack_pairs(h):
    n, h_dim = h.shape
    hp = h.reshape(n, h_dim // 32, 2, 16).transpose(0, 1, 3, 2)
    return jax.lax.bitcast_convert_type(hp.reshape(n, h_dim // 2, 2),
                                        jnp.uint32)


@jax.jit
def kernel(x, edge_index, edge_weight, node_graph_index,
           W0, b0, W1, b1, Wm1, bm1, Wm2, bm2):
    pad = EP - E
    row = jnp.concatenate([edge_index[0], jnp.zeros((pad,), jnp.int32)])
    col = jnp.concatenate([edge_index[1], jnp.zeros((pad,), jnp.int32)])
    ew = jnp.concatenate([edge_weight, jnp.zeros((pad,), jnp.float32)])
    row = row.reshape(NW, NCH, CH)
    col = col.reshape(NW, NCH, CH)
    ew = ew.reshape(NW, NCH, CH)

    degp = _sc_degree(row, ew)
    deg0 = degp[0, :N].reshape(N, 1)
    deg1 = degp[1, :N].reshape(N, 1)

    h0s, dinv = _tc_stage1(deg0, deg1, x, W0)

    p = _sc_edge64(_pack_pairs(h0s), row, col, ew)
    h1s = _tc_stage2(p[0, :N], p[1, :N], h0s, dinv, b0.reshape(1, H0), W1)

    q = _sc_edge32(_pack_pairs(h1s), row, col, ew)
    logits = _tc_stage3(q[0, :N], q[1, :N], h1s, dinv, b1.reshape(1, H1),
                        node_graph_index.reshape(N, 1),
                        Wm1, bm1.reshape(1, H_MLP), Wm2, bm2.reshape(1, NUM_CLASSES))
    return logits


# speedup vs baseline: 1.1480x; 1.1330x over previous
import functools

import jax
import jax.numpy as jnp
from jax import lax
from jax.experimental import pallas as pl
from jax.experimental.pallas import tpu as pltpu
from jax.experimental.pallas import tpu_sc as plsc

N = 10000
E = 320000
D_FEAT = 128
NUM_GRAPHS = 64
NUM_CLASSES = 10
H0 = 64
H1 = 32
H_MLP = 128

NCORE = 2
NSUB = 16
NW = NCORE * NSUB
NP = 10240
EP = 327680
EW_CNT = EP // NW
CH = 128
NCH = EW_CNT // CH
ZR = 128

_mesh = plsc.VectorSubcoreMesh(core_axis_name="c", subcore_axis_name="s")


@functools.partial(
    pl.kernel,
    out_type=jax.ShapeDtypeStruct((NCORE, NP), jnp.float32),
    mesh=_mesh,
    scratch_types=[
        pltpu.VMEM((NCH, CH), jnp.int32),
        pltpu.VMEM((NCH, CH), jnp.float32),
        pltpu.VMEM((NP // NSUB,), jnp.float32),
        pltpu.VMEM_SHARED((NP,), jnp.float32),
        pltpu.SemaphoreType.DMA,
    ],
)
def _sc_degree(row_hbm, ew_hbm, out_hbm, rowm, ewm, zbuf, acc, sem):
    c = lax.axis_index("c")
    s = lax.axis_index("s")
    wid = c * NSUB + s
    slab = NP // NSUB

    pltpu.sync_copy(row_hbm.at[wid], rowm)
    pltpu.sync_copy(ew_hbm.at[wid], ewm)

    def zb(i, _):
        zbuf[pl.ds(i * 16, 16)] = jnp.zeros((16,), jnp.float32)
        return 0

    lax.fori_loop(0, slab // 16, zb, 0)
    pltpu.sync_copy(zbuf, acc.at[pl.ds(s * slab, slab)])
    plsc.subcore_barrier()

    W = 8

    def _issue(g):
        pltpu.async_copy(ewm.at[g], acc.at[rowm.at[g]], sem, add=True)

    def _wait(g):
        pltpu.make_async_copy(ewm.at[g], acc.at[rowm.at[g]], sem).wait()

    for g in range(W):
        _issue(g)

    def deg_body(g, _):
        _issue(g)
        _wait(g - W)
        return 0

    lax.fori_loop(W, NCH, deg_body, 0)
    for g in range(NCH - W, NCH):
        _wait(g)
    plsc.subcore_barrier()
    pltpu.sync_copy(acc.at[pl.ds(s * slab, slab)], out_hbm.at[c, pl.ds(s * slab, slab)])


NBUF = 5
GA = 4
SBUF = 2
RND = 10


def _make_sc_edge(H):
    H2 = H // 2

    @functools.partial(
        pl.kernel,
        out_type=jax.ShapeDtypeStruct((NCORE, NP, H), jnp.float32),
        mesh=_mesh,
        scratch_types=[
            pltpu.VMEM((NCH, CH), jnp.int32),
            pltpu.VMEM((NCH, CH), jnp.int32),
            pltpu.VMEM((NCH, CH), jnp.float32),
            pltpu.VMEM((NBUF, CH, H2), jnp.uint32),
            pltpu.VMEM((SBUF, CH, H), jnp.float32),
            pltpu.VMEM((ZR, H), jnp.float32),
            pltpu.VMEM_SHARED((NP, H), jnp.float32),
        ] + [pltpu.SemaphoreType.DMA] * (NBUF + SBUF),
        compiler_params=pltpu.CompilerParams(use_tc_tiling_on_sc=False,
                                             needs_layout_passes=False),
    )
    def _sc_edge(h_hbm, row_hbm, col_hbm, ew_hbm, out_hbm, rowm, colm, ewm,
                 msg, msgf, zbuf, acc, *sems):
        gsem = sems[:NBUF]
        ssem = sems[NBUF:]
        c = lax.axis_index("c")
        s = lax.axis_index("s")
        wid = c * NSUB + s
        slab = NP // NSUB

        pltpu.sync_copy(row_hbm.at[wid], rowm)
        pltpu.sync_copy(col_hbm.at[wid], colm)
        pltpu.sync_copy(ew_hbm.at[wid], ewm)

        def zb(j, _):
            for k in range(H // 16):
                zbuf[j, pl.ds(k * 16, 16)] = jnp.zeros((16,), jnp.float32)
            return 0

        lax.fori_loop(0, ZR, zb, 0)

        for j in range(slab // ZR):
            pltpu.async_copy(zbuf, acc.at[pl.ds(s * slab + j * ZR, ZR), :],
                             ssem[0])
        for j in range(slab // ZR):
            pltpu.make_async_copy(
                zbuf, acc.at[pl.ds(s * slab + j * ZR, ZR), :], ssem[0]).wait()
        plsc.subcore_barrier()

        def issue_gather(g, b):
            pltpu.async_copy(h_hbm.at[colm.at[g]], msg.at[b], gsem[b])

        def wait_gather(g, b):
            pltpu.make_async_copy(h_hbm.at[colm.at[g]], msg.at[b],
                                  gsem[b]).wait()

        def issue_scatter(g, sb):
            pltpu.async_copy(msgf.at[sb], acc.at[rowm.at[g]], ssem[sb],
                             add=True)

        def wait_scatter(g, sb):
            pltpu.make_async_copy(msgf.at[sb], acc.at[rowm.at[g]],
                                  ssem[sb]).wait()

        hi_mask = jnp.uint32(0xFFFF0000)

        def scale(g, b, sb):
            def body(j, _):
                w16 = ewm[g, pl.ds(j * 16, 16)]
                for t in range(16):
                    w = w16[t]
                    e = j * 16 + t
                    for k in range(H2 // 16):
                        cvec = msg[b, e, pl.ds(k * 16, 16)]
                        lo = plsc.bitcast(cvec << 16, jnp.float32)
                        hi = plsc.bitcast(cvec & hi_mask, jnp.float32)
                        msgf[sb, e, pl.ds(k * 32, 16)] = lo * w
                        msgf[sb, e, pl.ds(k * 32 + 16, 16)] = hi * w
                return 0

            lax.fori_loop(0, CH // 16, body, 0)

        def step(g, b, sb, do_wait_scatter, do_issue_gather, bf):
            if do_wait_scatter:
                wait_scatter(g - SBUF, sb)
            wait_gather(g, b)
            scale(g, b, sb)
            issue_scatter(g, sb)
            if do_issue_gather:
                issue_gather(g + GA, bf)

        for g in range(GA):
            issue_gather(g, g)
        for g in range(RND):
            step(g, g % NBUF, g % SBUF, g >= SBUF, True, (g + GA) % NBUF)

        def round_body(i, _):
            g0 = i * RND
            for k in range(RND):
                step(g0 + k, k % NBUF, k % SBUF, True, True,
                     (k + GA) % NBUF)
            return 0

        lax.fori_loop(1, NCH // RND - 1, round_body, 0)

        for g in range(NCH - RND, NCH):
            step(g, g % NBUF, g % SBUF, True, g + GA < NCH, (g + GA) % NBUF)
        for g in range(NCH - SBUF, NCH):
            wait_scatter(g, g % SBUF)
        plsc.subcore_barrier()
        pltpu.sync_copy(acc.at[pl.ds(s * slab, slab), :],
                        out_hbm.at[c, pl.ds(s * slab, slab), :])

    return _sc_edge


_sc_edge64 = _make_sc_edge(H0)
_sc_edge32 = _make_sc_edge(H1)


def _pack_in_kernel(h):
    n, h_dim = h.shape
    b32 = lax.bitcast_convert_type(h, jnp.uint32)
    r = (b32 + jnp.uint32(0x7FFF) + ((b32 >> 16) & jnp.uint32(1))) >> 16
    los = [r[:, 32 * s:32 * s + 16] for s in range(h_dim // 32)]
    his = [r[:, 32 * s + 16:32 * s + 32] for s in range(h_dim // 32)]
    lo = jnp.concatenate(los, axis=1) if len(los) > 1 else los[0]
    hi = jnp.concatenate(his, axis=1) if len(his) > 1 else his[0]
    return lo | (hi << 16)


def _tc_stage1_body(deg0_ref, deg1_ref, x_ref, w0_ref, h0s_ref, h0u_ref,
                    dinv_ref):
    deg = deg0_ref[...] + deg1_ref[...] + 1.0
    dinv = lax.rsqrt(jnp.maximum(deg, 1e-12))
    dinv_ref[...] = dinv
    h0 = jnp.dot(x_ref[...], w0_ref[...], preferred_element_type=jnp.float32)
    h0s = h0 * dinv
    h0s_ref[...] = h0s
    h0u_ref[...] = _pack_in_kernel(h0s)


def _tc_stage1(deg0, deg1, x, W0):
    return pl.pallas_call(
        _tc_stage1_body,
        out_shape=(
            jax.ShapeDtypeStruct((N, H0), jnp.float32),
            jax.ShapeDtypeStruct((N, H0 // 2), jnp.uint32),
            jax.ShapeDtypeStruct((N, 1), jnp.float32),
        ),
    )(deg0, deg1, x, W0)


def _tc_stage2_body(p0_ref, p1_ref, h0s_ref, dinv_ref, b0_ref, w1_ref,
                    h1s_ref, h1u_ref):
    dinv = dinv_ref[...]
    a1 = jnp.maximum(
        dinv * (p0_ref[...] + p1_ref[...] + h0s_ref[...]) + b0_ref[...], 0.0)
    h1 = jnp.dot(a1, w1_ref[...], preferred_element_type=jnp.float32)
    h1s = h1 * dinv
    h1s_ref[...] = h1s
    h1u_ref[...] = _pack_in_kernel(h1s)


def _tc_stage2(p0, p1, h0s, dinv, b0, W1):
    return pl.pallas_call(
        _tc_stage2_body,
        out_shape=(
            jax.ShapeDtypeStruct((N, H1), jnp.float32),
            jax.ShapeDtypeStruct((N, H1 // 2), jnp.uint32),
        ),
    )(p0, p1, h0s, dinv, b0, W1)


def _tc_stage3_body(q0_ref, q1_ref, h1s_ref, dinv_ref, b1_ref, ngi_ref,
                    wm1_ref, bm1_ref, wm2_ref, bm2_ref, out_ref):
    dinv = dinv_ref[...]
    a2 = jnp.maximum(
        dinv * (q0_ref[...] + q1_ref[...] + h1s_ref[...]) + b1_ref[...], 0.0)
    gids = lax.broadcasted_iota(jnp.int32, (N, NUM_GRAPHS), 1)
    onehot = (ngi_ref[...] == gids).astype(jnp.float32)
    pooled = lax.dot_general(onehot, a2, (((0,), (0,)), ((), ())),
                             preferred_element_type=jnp.float32)
    h2 = jnp.maximum(
        jnp.dot(pooled, wm1_ref[...], preferred_element_type=jnp.float32)
        + bm1_ref[...], 0.0)
    out_ref[...] = jnp.dot(h2, wm2_ref[...],
                           preferred_element_type=jnp.float32) + bm2_ref[...]


def _tc_stage3(q0, q1, h1s, dinv, b1, ngi, Wm1, bm1, Wm2, bm2):
    return pl.pallas_call(
        _tc_stage3_body,
        out_shape=jax.ShapeDtypeStruct((NUM_GRAPHS, NUM_CLASSES), jnp.float32),
    )(q0, q1, h1s, dinv, b1, ngi, Wm1, bm1, Wm2, bm2)


@jax.jit
def kernel(x, edge_index, edge_weight, node_graph_index,
           W0, b0, W1, b1, Wm1, bm1, Wm2, bm2):
    pad = EP - E
    row = jnp.concatenate([edge_index[0], jnp.zeros((pad,), jnp.int32)])
    col = jnp.concatenate([edge_index[1], jnp.zeros((pad,), jnp.int32)])
    ew = jnp.concatenate([edge_weight, jnp.zeros((pad,), jnp.float32)])
    row = row.reshape(NW, NCH, CH)
    col = col.reshape(NW, NCH, CH)
    ew = ew.reshape(NW, NCH, CH)

    degp = _sc_degree(row, ew)
    deg0 = degp[0, :N].reshape(N, 1)
    deg1 = degp[1, :N].reshape(N, 1)

    h0s, h0u, dinv = _tc_stage1(deg0, deg1, x, W0)

    p = _sc_edge64(h0u, row, col, ew)
    h1s, h1u = _tc_stage2(p[0, :N], p[1, :N], h0s, dinv, b0.reshape(1, H0), W1)

    q = _sc_edge32(h1u, row, col, ew)
    logits = _tc_stage3(q[0, :N], q[1, :N], h1s, dinv, b1.reshape(1, H1),
                        node_graph_index.reshape(N, 1),
                        Wm1, bm1.reshape(1, H_MLP), Wm2, bm2.reshape(1, NUM_CLASSES))
    return logits
